# baseline (speedup 1.0000x reference)
import functools

import jax
import jax.numpy as jnp
from jax.experimental import pallas as pl
from jax.experimental.pallas import tpu as pltpu
from jax.experimental.pallas import tpu_sc as plsc

_VOCAB_BLOCK = 5000
_GATHER_WINDOW = 128
_ENC_W = 32
_PROBS_W = 16


def _table_body(emb_ref, W1_ref, b1_ref, W2_ref, b2_ref, W3_ref, b3_ref,
                We_ref, be_ref, Wc_ref, bc_ref, enc_ref, probs_ref):
    x = emb_ref[...]
    x = jnp.maximum(
        jnp.dot(x, W1_ref[...], preferred_element_type=jnp.float32) + b1_ref[...], 0.0)
    x = jnp.maximum(
        jnp.dot(x, W2_ref[...], preferred_element_type=jnp.float32) + b2_ref[...], 0.0)
    x = jnp.maximum(
        jnp.dot(x, W3_ref[...], preferred_element_type=jnp.float32) + b3_ref[...], 0.0)
    enc = jnp.dot(x, We_ref[...], preferred_element_type=jnp.float32) + be_ref[...]
    enc_ref[...] = enc
    logits = jnp.dot(enc, Wc_ref[...], preferred_element_type=jnp.float32) + bc_ref[...]
    z = logits - jnp.max(logits, axis=-1, keepdims=True)
    e = jnp.exp(z)
    probs_ref[...] = e / jnp.sum(e, axis=-1, keepdims=True)


def _mlp_table(emb, W1, b1, W2, b2, W3, b3, We, be, Wc_pad, bc_pad):
    vocab, embed = emb.shape
    grid = vocab // _VOCAB_BLOCK

    def full(shape):
        return pl.BlockSpec(shape, lambda i: (0, 0))

    return pl.pallas_call(
        _table_body,
        grid=(grid,),
        in_specs=[
            pl.BlockSpec((_VOCAB_BLOCK, embed), lambda i: (i, 0)),
            full(W1.shape), full(b1.shape),
            full(W2.shape), full(b2.shape),
            full(W3.shape), full(b3.shape),
            full(We.shape), full(be.shape),
            full(Wc_pad.shape), full(bc_pad.shape),
        ],
        out_specs=[
            pl.BlockSpec((_VOCAB_BLOCK, _ENC_W), lambda i: (i, 0)),
            pl.BlockSpec((_VOCAB_BLOCK, _PROBS_W), lambda i: (i, 0)),
        ],
        out_shape=[
            jax.ShapeDtypeStruct((vocab, _ENC_W), jnp.float32),
            jax.ShapeDtypeStruct((vocab, _PROBS_W), jnp.float32),
        ],
        compiler_params=pltpu.CompilerParams(dimension_semantics=("parallel",)),
    )(emb, W1, b1, W2, b2, W3, b3, We, be, Wc_pad, bc_pad)


def _sc_gather(enc_table, probs_table, ids_2d):
    n = ids_2d.shape[1]
    mesh = plsc.VectorSubcoreMesh(core_axis_name="core", subcore_axis_name="subcore")

    @functools.partial(
        pl.kernel,
        out_type=(
            jax.ShapeDtypeStruct((n, _ENC_W), jnp.float32),
            jax.ShapeDtypeStruct((n, _PROBS_W), jnp.float32),
        ),
        mesh=mesh,
        compiler_params=pltpu.CompilerParams(use_tc_tiling_on_sc=False),
    )
    def k(enc_hbm, probs_hbm, i_hbm, oe_hbm, op_hbm):
        def body(i_vmem, oe_vmem, op_vmem):
            pltpu.sync_copy(enc_hbm.at[i_vmem.at[0]], oe_vmem)
            pltpu.sync_copy(probs_hbm.at[i_vmem.at[0]], op_vmem)

        pltpu.emit_pipeline(
            body,
            grid=(n // _GATHER_WINDOW,),
            in_specs=[pl.BlockSpec((1, _GATHER_WINDOW), index_map=lambda i: (0, i))],
            out_specs=[
                pl.BlockSpec((_GATHER_WINDOW, _ENC_W), index_map=lambda i: (i, 0)),
                pl.BlockSpec((_GATHER_WINDOW, _PROBS_W), index_map=lambda i: (i, 0)),
            ],
            core_axis_name=("core", "subcore"),
            dimension_semantics=(pltpu.PARALLEL,),
        )(i_hbm, oe_hbm, op_hbm)

    return k(enc_table, probs_table, ids_2d)


def kernel(input_word_ids, emb, W1, b1, W2, b2, W3, b3, We, be, Wc, bc):
    batch, seq = input_word_ids.shape
    n = batch * seq
    cats = Wc.shape[1]

    ids_2d = input_word_ids.astype(jnp.int32).reshape(1, n)
    Wc_pad = jnp.pad(Wc, ((0, 0), (0, _PROBS_W - cats)))
    bc_pad = jnp.pad(bc, (0, _PROBS_W - cats), constant_values=-1e30)

    enc_table, probs_table = _mlp_table(
        emb.astype(jnp.bfloat16), W1.astype(jnp.bfloat16),
        b1.reshape(1, -1), W2, b2.reshape(1, -1),
        W3, b3.reshape(1, -1), We, be.reshape(1, -1),
        Wc_pad, bc_pad.reshape(1, -1))

    enc_g, probs_g = _sc_gather(enc_table, probs_table, ids_2d)
    enc = enc_g.reshape(batch, seq, _ENC_W)
    probs = probs_g[:, :cats].reshape(batch, seq, cats)
    return enc, probs


# speedup vs baseline: 1.8100x; 1.8100x over previous
import functools

import jax
import jax.numpy as jnp
from jax.experimental import pallas as pl
from jax.experimental.pallas import tpu as pltpu
from jax.experimental.pallas import tpu_sc as plsc

_VOCAB_BLOCK = 4096
_GATHER_WINDOW = 128
_ENC_W = 32
_PROBS_W = 16


def _table_body(embT_ref, W1T_ref, b1_ref, W2T_ref, b2_ref, W3T_ref, b3_ref,
                We_ref, be_ref, Wc_ref, bc_ref, enc_ref, probs_ref):
    xt = embT_ref[...].astype(jnp.bfloat16)
    x1 = jnp.maximum(
        jnp.dot(W1T_ref[...], xt, preferred_element_type=jnp.float32)
        + b1_ref[...], 0.0)
    x2 = jnp.maximum(
        jnp.dot(W2T_ref[...], x1, preferred_element_type=jnp.float32)
        + b2_ref[...], 0.0)
    x3 = jnp.maximum(
        jnp.dot(W3T_ref[...], x2, preferred_element_type=jnp.float32)
        + b3_ref[...], 0.0)
    enc = jax.lax.dot_general(
        x3, We_ref[...], (((0,), (0,)), ((), ())),
        preferred_element_type=jnp.float32) + be_ref[...]
    enc_ref[...] = enc
    logits = jnp.dot(enc, Wc_ref[...], preferred_element_type=jnp.float32) + bc_ref[...]
    z = logits - jnp.max(logits, axis=-1, keepdims=True)
    e = jnp.exp(z)
    probs_ref[...] = e / jnp.sum(e, axis=-1, keepdims=True)


def _mlp_table(embT, W1T, b1, W2T, b2, W3T, b3, We, be, Wc_pad, bc_pad):
    embed, vocab = embT.shape
    grid = (vocab + _VOCAB_BLOCK - 1) // _VOCAB_BLOCK

    def full(shape):
        return pl.BlockSpec(shape, lambda i: (0, 0))

    return pl.pallas_call(
        _table_body,
        grid=(grid,),
        in_specs=[
            pl.BlockSpec((embed, _VOCAB_BLOCK), lambda i: (0, i)),
            full(W1T.shape), full(b1.shape),
            full(W2T.shape), full(b2.shape),
            full(W3T.shape), full(b3.shape),
            full(We.shape), full(be.shape),
            full(Wc_pad.shape), full(bc_pad.shape),
        ],
        out_specs=[
            pl.BlockSpec((_VOCAB_BLOCK, _ENC_W), lambda i: (i, 0)),
            pl.BlockSpec((_VOCAB_BLOCK, _PROBS_W), lambda i: (i, 0)),
        ],
        out_shape=[
            jax.ShapeDtypeStruct((vocab, _ENC_W), jnp.float32),
            jax.ShapeDtypeStruct((vocab, _PROBS_W), jnp.float32),
        ],
        compiler_params=pltpu.CompilerParams(dimension_semantics=("parallel",)),
    )(embT, W1T, b1, W2T, b2, W3T, b3, We, be, Wc_pad, bc_pad)


def _sc_gather(enc_table, probs_table, ids_2d):
    n = ids_2d.shape[1]
    mesh = plsc.VectorSubcoreMesh(core_axis_name="core", subcore_axis_name="subcore")

    @functools.partial(
        pl.kernel,
        out_type=(
            jax.ShapeDtypeStruct((n, _ENC_W), jnp.float32),
            jax.ShapeDtypeStruct((n, _PROBS_W), jnp.float32),
        ),
        mesh=mesh,
        compiler_params=pltpu.CompilerParams(use_tc_tiling_on_sc=False),
    )
    def k(enc_hbm, probs_hbm, i_hbm, oe_hbm, op_hbm):
        def body(i_vmem, oe_vmem, op_vmem):
            pltpu.sync_copy(enc_hbm.at[i_vmem.at[0]], oe_vmem)
            pltpu.sync_copy(probs_hbm.at[i_vmem.at[0]], op_vmem)

        pltpu.emit_pipeline(
            body,
            grid=(n // _GATHER_WINDOW,),
            in_specs=[pl.BlockSpec((1, _GATHER_WINDOW), index_map=lambda i: (0, i))],
            out_specs=[
                pl.BlockSpec((_GATHER_WINDOW, _ENC_W), index_map=lambda i: (i, 0)),
                pl.BlockSpec((_GATHER_WINDOW, _PROBS_W), index_map=lambda i: (i, 0)),
            ],
            core_axis_name=("core", "subcore"),
            dimension_semantics=(pltpu.PARALLEL,),
        )(i_hbm, oe_hbm, op_hbm)

    return k(enc_table, probs_table, ids_2d)


def kernel(input_word_ids, emb, W1, b1, W2, b2, W3, b3, We, be, Wc, bc):
    batch, seq = input_word_ids.shape
    n = batch * seq
    cats = Wc.shape[1]

    ids_2d = input_word_ids.astype(jnp.int32).reshape(1, n)
    Wc_pad = jnp.pad(Wc, ((0, 0), (0, _PROBS_W - cats)))
    bc_pad = jnp.pad(bc, (0, _PROBS_W - cats), constant_values=-1e30)

    enc_table, probs_table = _mlp_table(
        emb.T, W1.T.astype(jnp.bfloat16),
        b1.reshape(-1, 1), W2.T, b2.reshape(-1, 1),
        W3.T, b3.reshape(-1, 1), We, be.reshape(1, -1),
        Wc_pad, bc_pad.reshape(1, -1))

    enc_g, probs_g = _sc_gather(enc_table, probs_table, ids_2d)
    enc = enc_g.reshape(batch, seq, _ENC_W)
    probs = probs_g[:, :cats].reshape(batch, seq, cats)
    return enc, probs
